# baseline (speedup 1.0000x reference)
import functools

import jax
import jax.numpy as jnp
from jax import lax
from jax.experimental import pallas as pl
from jax.experimental.pallas import tpu as pltpu
from jax.experimental.pallas import tpu_sc as plsc

INPUT_DIM = 4096
HIDDEN_DIMS = [8192, 8192]
OUTPUT_DIM = 4096
FANIN = 64
LAYER_DIMS = [INPUT_DIM] + HIDDEN_DIMS + [OUTPUT_DIM]
LAYER_INDICES = [0]
for _d in LAYER_DIMS:
    LAYER_INDICES.append(LAYER_INDICES[-1] + _d)
TOTAL_ROWS = sum(LAYER_DIMS[1:])

NC = 2
NS = 16
LANES = 16
PAD = 72

_LAYERS = []
for _i in range(1, len(LAYER_DIMS)):
    _LAYERS.append((LAYER_DIMS[_i], LAYER_INDICES[_i - 1],
                    LAYER_INDICES[_i] - INPUT_DIM,
                    LAYER_INDICES[_i] - INPUT_DIM))

_MAX_NPT = max(n for n, _, _, _ in _LAYERS) // NS


def _build_net_kernel():
    mesh = plsc.VectorSubcoreMesh(core_axis_name="c", subcore_axis_name="s")
    out_type = [
        jax.ShapeDtypeStruct((NC, HIDDEN_DIMS[0]), jnp.float32),
        jax.ShapeDtypeStruct((NC, HIDDEN_DIMS[1]), jnp.float32),
        jax.ShapeDtypeStruct((OUTPUT_DIM,), jnp.float32),
    ]

    @functools.partial(
        pl.kernel,
        out_type=out_type,
        mesh=mesh,
        compiler_params=pltpu.CompilerParams(needs_layout_passes=False,
                                             use_tc_tiling_on_sc=False,
                                             disable_bounds_checks=True),
        scratch_types=[
            pltpu.VMEM((_MAX_NPT, PAD), jnp.int32),
            pltpu.VMEM((_MAX_NPT, PAD), jnp.float32),
            pltpu.VMEM((HIDDEN_DIMS[0],), jnp.float32),
            pltpu.VMEM((TOTAL_ROWS // NS,), jnp.float32),
            pltpu.VMEM((_MAX_NPT,), jnp.float32),
            pltpu.SemaphoreType.DMA,
            pltpu.SemaphoreType.DMA,
            pltpu.SemaphoreType.DMA,
            pltpu.SemaphoreType.DMA,
            pltpu.SemaphoreType.DMA,
            pltpu.SemaphoreType.DMA,
        ],
    )
    def net(x_hbm, src_hbm, w_hbm, b_hbm, st1, st2, out_hbm,
            src_v, w_v, acts_v, bias_v, out_v, s0, s1, s2, s3, s4, s5):
        c = lax.axis_index("c")
        s = lax.axis_index("s")
        lane = lax.iota(jnp.int32, LANES)
        zero = jnp.zeros((LANES,), jnp.float32)
        ssrc = (s0, s1)
        sw = (s2, s3)

        def fire_edges(row0, nrows, dst0=0, par=0):
            c1 = pltpu.async_copy(
                src_hbm.at[pl.ds(row0, nrows), :],
                src_v.at[pl.ds(dst0, nrows), pl.ds(0, FANIN)], ssrc[par])
            c2 = pltpu.async_copy(
                w_hbm.at[pl.ds(row0, nrows), :],
                w_v.at[pl.ds(dst0, nrows), pl.ds(0, FANIN)], sw[par])
            return c1, c2

        def compute_layer(nrows, prev_start, bias_base, row_base=0):
            def group(g, _):
                rows = row_base + g * LANES + lane

                def jblock(t, accs):
                    a0, a1, a2, a3 = accs
                    accs = [a0, a1, a2, a3]
                    for u in range(8):
                        cols = jnp.full((LANES,), 0, jnp.int32) + (t * 8 + u)
                        si = plsc.load_gather(src_v, [rows, cols])
                        wv = plsc.load_gather(w_v, [rows, cols])
                        av = plsc.load_gather(acts_v, [si - prev_start])
                        accs[u % 4] = accs[u % 4] + av * wv
                    return tuple(accs)

                accs = lax.fori_loop(0, FANIN // 8, jblock,
                                     (zero, zero, zero, zero))
                acc = (accs[0] + accs[1]) + (accs[2] + accs[3])
                nb = row_base + g * LANES
                b = bias_v[pl.ds(bias_base + nb, LANES)]
                out_v[pl.ds(nb, LANES)] = jnp.maximum(acc + b, 0.0)
                return 0

            lax.fori_loop(0, nrows // LANES, group, 0)

        n1, ps1, er1, bb1 = _LAYERS[0]
        npt1 = n1 // NS
        n2, ps2, er2, bb2 = _LAYERS[1]
        npt2 = n2 // NS
        n3, ps3, er3, bb3 = _LAYERS[2]
        npt3 = n3 // NS
        cx = pltpu.async_copy(x_hbm, acts_v.at[pl.ds(0, INPUT_DIM)], s4)
        cb1 = pltpu.async_copy(b_hbm.at[pl.ds(bb1 + s * npt1, npt1)],
                               bias_v.at[pl.ds(0, npt1)], s5)
        cb2 = pltpu.async_copy(b_hbm.at[pl.ds(bb2 + s * npt2, npt2)],
                               bias_v.at[pl.ds(npt1, npt2)], s5)
        cb3 = pltpu.async_copy(b_hbm.at[pl.ds(bb3 + s * npt3, npt3)],
                               bias_v.at[pl.ds(npt1 + npt2, npt3)], s5)
        q = npt1 // 4
        r1 = er1 + s * npt1
        pq = [fire_edges(r1, q, 0, 0), fire_edges(r1 + q, q, q, 1)]
        cx.wait()
        cb1.wait()
        cb2.wait()
        cb3.wait()
        for k in range(4):
            pq[k][0].wait()
            pq[k][1].wait()
            if k + 2 < 4:
                pq.append(fire_edges(r1 + (k + 2) * q, q, (k + 2) * q, k % 2))
            compute_layer(q, ps1, 0, row_base=k * q)

        p0 = fire_edges(er2 + s * npt2, npt2)
        co = pltpu.async_copy(out_v, st1.at[c, pl.ds(s * npt1, npt1)], s4)
        co.wait()
        plsc.subcore_barrier()
        ca = pltpu.async_copy(st1.at[c], acts_v, s4)
        ca.wait()
        p0[0].wait()
        p0[1].wait()
        compute_layer(npt2, ps2, npt1)

        p0 = fire_edges(er3 + s * npt3, npt3)
        co = pltpu.async_copy(out_v, st2.at[c, pl.ds(s * npt2, npt2)], s4)
        co.wait()
        plsc.subcore_barrier()
        ca = pltpu.async_copy(st2.at[c], acts_v, s4)
        ca.wait()
        p0[0].wait()
        p0[1].wait()
        compute_layer(npt3, ps3, npt1 + npt2)

        @pl.when(c == 0)
        def _():
            pltpu.async_copy(out_v.at[pl.ds(0, npt3)],
                             out_hbm.at[pl.ds(s * npt3, npt3)], s4).wait()

    return net


_NET = _build_net_kernel()


def kernel(x, edge_index, weights, bias):
    src2d = edge_index[0].reshape(TOTAL_ROWS, FANIN)
    w2d = weights.reshape(TOTAL_ROWS, FANIN)
    _, _, out = _NET(x, src2d, w2d, bias)
    return out


# speedup vs baseline: 1.0150x; 1.0150x over previous
import functools

import jax
import jax.numpy as jnp
from jax import lax
from jax.experimental import pallas as pl
from jax.experimental.pallas import tpu as pltpu
from jax.experimental.pallas import tpu_sc as plsc

INPUT_DIM = 4096
HIDDEN_DIMS = [8192, 8192]
OUTPUT_DIM = 4096
FANIN = 64
LAYER_DIMS = [INPUT_DIM] + HIDDEN_DIMS + [OUTPUT_DIM]
LAYER_INDICES = [0]
for _d in LAYER_DIMS:
    LAYER_INDICES.append(LAYER_INDICES[-1] + _d)
TOTAL_ROWS = sum(LAYER_DIMS[1:])

NC = 2
NS = 16
LANES = 16
PAD = 72

_LAYERS = []
for _i in range(1, len(LAYER_DIMS)):
    _LAYERS.append((LAYER_DIMS[_i], LAYER_INDICES[_i - 1],
                    LAYER_INDICES[_i] - INPUT_DIM,
                    LAYER_INDICES[_i] - INPUT_DIM))

_MAX_NPT = max(n for n, _, _, _ in _LAYERS) // NS


def _build_net_kernel():
    mesh = plsc.VectorSubcoreMesh(core_axis_name="c", subcore_axis_name="s")
    out_type = [
        jax.ShapeDtypeStruct((NC, HIDDEN_DIMS[0]), jnp.float32),
        jax.ShapeDtypeStruct((NC, HIDDEN_DIMS[1]), jnp.float32),
        jax.ShapeDtypeStruct((OUTPUT_DIM,), jnp.float32),
    ]

    @functools.partial(
        pl.kernel,
        out_type=out_type,
        mesh=mesh,
        compiler_params=pltpu.CompilerParams(needs_layout_passes=False,
                                             use_tc_tiling_on_sc=False,
                                             disable_bounds_checks=True),
        scratch_types=[
            pltpu.VMEM((_MAX_NPT, PAD), jnp.int32),
            pltpu.VMEM((_MAX_NPT, PAD), jnp.float32),
            pltpu.VMEM((HIDDEN_DIMS[0],), jnp.float32),
            pltpu.VMEM((TOTAL_ROWS // NS,), jnp.float32),
            pltpu.VMEM((_MAX_NPT,), jnp.float32),
            pltpu.SemaphoreType.DMA,
            pltpu.SemaphoreType.DMA,
            pltpu.SemaphoreType.DMA,
            pltpu.SemaphoreType.DMA,
            pltpu.SemaphoreType.DMA,
            pltpu.SemaphoreType.DMA,
        ],
    )
    def net(x_hbm, src_hbm, w_hbm, b_hbm, st1, st2, out_hbm,
            src_v, w_v, acts_v, bias_v, out_v, s0, s1, s2, s3, s4, s5):
        c = lax.axis_index("c")
        s = lax.axis_index("s")
        lane = lax.iota(jnp.int32, LANES)
        zero = jnp.zeros((LANES,), jnp.float32)
        ssrc = (s0, s1)
        sw = (s2, s3)

        def fire_edges(row0, nrows, dst0=0, par=0):
            c1 = pltpu.async_copy(
                src_hbm.at[pl.ds(row0, nrows), :],
                src_v.at[pl.ds(dst0, nrows), pl.ds(0, FANIN)], ssrc[par])
            c2 = pltpu.async_copy(
                w_hbm.at[pl.ds(row0, nrows), :],
                w_v.at[pl.ds(dst0, nrows), pl.ds(0, FANIN)], sw[par])
            return c1, c2

        def compute_layer(nrows, prev_start, bias_base, row_base=0):
            def group(g, _):
                rows = row_base + g * LANES + lane

                def jblock(t, accs):
                    a0, a1, a2, a3 = accs
                    accs = [a0, a1, a2, a3]
                    for u in range(8):
                        cols = jnp.full((LANES,), 0, jnp.int32) + (t * 8 + u)
                        si = plsc.load_gather(src_v, [rows, cols])
                        wv = plsc.load_gather(w_v, [rows, cols])
                        av = plsc.load_gather(acts_v, [si - prev_start])
                        accs[u % 4] = accs[u % 4] + av * wv
                    return tuple(accs)

                accs = lax.fori_loop(0, FANIN // 8, jblock,
                                     (zero, zero, zero, zero))
                acc = (accs[0] + accs[1]) + (accs[2] + accs[3])
                nb = row_base + g * LANES
                b = bias_v[pl.ds(bias_base + nb, LANES)]
                out_v[pl.ds(nb, LANES)] = jnp.maximum(acc + b, 0.0)
                return 0

            lax.fori_loop(0, nrows // LANES, group, 0)

        n1, ps1, er1, bb1 = _LAYERS[0]
        npt1 = n1 // NS
        n2, ps2, er2, bb2 = _LAYERS[1]
        npt2 = n2 // NS
        n3, ps3, er3, bb3 = _LAYERS[2]
        npt3 = n3 // NS
        cx = pltpu.async_copy(x_hbm, acts_v.at[pl.ds(0, INPUT_DIM)], s4)
        cb1 = pltpu.async_copy(b_hbm.at[pl.ds(bb1 + s * npt1, npt1)],
                               bias_v.at[pl.ds(0, npt1)], s5)
        cb2 = pltpu.async_copy(b_hbm.at[pl.ds(bb2 + s * npt2, npt2)],
                               bias_v.at[pl.ds(npt1, npt2)], s5)
        cb3 = pltpu.async_copy(b_hbm.at[pl.ds(bb3 + s * npt3, npt3)],
                               bias_v.at[pl.ds(npt1 + npt2, npt3)], s5)
        half = npt1 // 2
        pa = fire_edges(er1 + s * npt1, half, 0, 0)
        pb = fire_edges(er1 + s * npt1 + half, half, half, 1)
        cx.wait()
        cb1.wait()
        cb2.wait()
        cb3.wait()
        pa[0].wait()
        pa[1].wait()
        compute_layer(half, ps1, 0)
        pb[0].wait()
        pb[1].wait()
        compute_layer(half, ps1, 0, row_base=half)

        p0 = fire_edges(er2 + s * npt2, npt2)
        co = pltpu.async_copy(out_v, st1.at[c, pl.ds(s * npt1, npt1)], s4)
        co.wait()
        plsc.subcore_barrier()
        ca = pltpu.async_copy(st1.at[c], acts_v, s4)
        ca.wait()
        p0[0].wait()
        p0[1].wait()
        compute_layer(npt2, ps2, npt1)

        p0 = fire_edges(er3 + s * npt3, npt3)
        co = pltpu.async_copy(out_v, st2.at[c, pl.ds(s * npt2, npt2)], s4)
        co.wait()
        plsc.subcore_barrier()
        ca = pltpu.async_copy(st2.at[c], acts_v, s4)
        ca.wait()
        p0[0].wait()
        p0[1].wait()
        compute_layer(npt3, ps3, npt1 + npt2)

        @pl.when(c == 0)
        def _():
            pltpu.async_copy(out_v.at[pl.ds(0, npt3)],
                             out_hbm.at[pl.ds(s * npt3, npt3)], s4).wait()

    return net


_NET = _build_net_kernel()


def kernel(x, edge_index, weights, bias):
    src2d = edge_index[0].reshape(TOTAL_ROWS, FANIN)
    w2d = weights.reshape(TOTAL_ROWS, FANIN)
    _, _, out = _NET(x, src2d, w2d, bias)
    return out
